# baseline (speedup 1.0000x reference)
import functools

import jax
import jax.numpy as jnp
from jax import lax
from jax.experimental import pallas as pl
from jax.experimental.pallas import tpu as pltpu
from jax.experimental.pallas import tpu_sc as plsc

_B = 1024
_D = 64
_N = 100000
_INV_TEMP = 20.0
_SHIFT = 20.0
_TILE = 4000
_STEPS = _N // _TILE
_LOG2E = 1.4426950408889634
_C = _SHIFT * _LOG2E

_NC = 2
_NS = 16
_NW = _NC * _NS
_BPW = _B // _NW


def _sc_gather_blocks(table, idx):
    mesh = plsc.ScalarSubcoreMesh(axis_name="c", num_cores=_NC)
    bpc = _B // _NC

    @functools.partial(
        pl.kernel,
        mesh=mesh,
        out_type=jax.ShapeDtypeStruct((8 * _B, _D), jnp.bfloat16),
        scratch_types=[
            pltpu.SMEM((bpc,), jnp.int32),
            pltpu.SemaphoreType.DMA,
        ],
    )
    def gather_k(table_hbm, idx_hbm, out_hbm, idx_s, sem):
        base = lax.axis_index("c") * bpc
        pltpu.async_copy(idx_hbm.at[pl.ds(base, bpc)], idx_s, sem).wait()

        @pl.loop(0, bpc)
        def _(i):
            tb = pl.multiple_of((idx_s[i] >> 3) << 3, 8)
            pltpu.async_copy(table_hbm.at[pl.ds(tb, 8)],
                             out_hbm.at[pl.ds((base + i) * 8, 8)], sem)

        @pl.loop(0, bpc)
        def _(i):
            pltpu.make_async_copy(table_hbm.at[pl.ds(0, 8)],
                                  out_hbm.at[pl.ds(0, 8)], sem).wait()

    return gather_k(table, idx)


def _tc_convert_kernel(f_ref, o_ref):
    o_ref[...] = f_ref[...].astype(jnp.bfloat16)


def _tc_convert(features):
    return pl.pallas_call(
        _tc_convert_kernel,
        grid=(_STEPS,),
        in_specs=[pl.BlockSpec((_TILE, _D), lambda k: (k, 0))],
        out_specs=pl.BlockSpec((_TILE, _D), lambda k: (k, 0)),
        out_shape=jax.ShapeDtypeStruct((_N, _D), jnp.bfloat16),
        compiler_params=pltpu.CompilerParams(
            dimension_semantics=("arbitrary",)),
    )(features)


def _tc_main_kernel(x_ref, f_ref, s_ref, xs_ref):
    k = pl.program_id(0)

    @pl.when(k == 0)
    def _():
        x = x_ref[...]
        nrm = jnp.sqrt(jnp.sum(x * x, axis=1, keepdims=True))
        xn = x / jnp.maximum(nrm, 1e-12)
        xs_ref[...] = (xn * (_INV_TEMP * _LOG2E)).astype(jnp.bfloat16)
        s_ref[...] = jnp.zeros_like(s_ref)

    logits2 = lax.dot_general(
        xs_ref[...], f_ref[...],
        (((1,), (1,)), ((), ())),
        preferred_element_type=jnp.float32)
    s_ref[...] += jnp.sum(jnp.exp2(logits2 - _C), axis=1, keepdims=True)


def _tc_main(inputs, features, interpret=False):
    return pl.pallas_call(
        _tc_main_kernel,
        grid=(_STEPS,),
        in_specs=[
            pl.BlockSpec((_B, _D), lambda k: (0, 0)),
            pl.BlockSpec((_TILE, _D), lambda k: (k, 0)),
        ],
        out_specs=pl.BlockSpec((_B, 1), lambda k: (0, 0)),
        out_shape=jax.ShapeDtypeStruct((_B, 1), jnp.float32),
        scratch_shapes=[
            pltpu.VMEM((_B, _D), jnp.bfloat16),
        ],
        compiler_params=pltpu.CompilerParams(
            dimension_semantics=("arbitrary",)),
        interpret=interpret,
    )(inputs, features)


def _tc_combine_kernel(x_ref, g_ref, t_ref, s_ref, out_ref):
    x = x_ref[...]
    nrm = jnp.sqrt(jnp.sum(x * x, axis=1, keepdims=True))
    xn = x / jnp.maximum(nrm, 1e-12)
    blocks = jnp.reshape(g_ref[...].astype(jnp.float32), (_B, 8, _D))
    r = jnp.reshape(t_ref[...] & 7, (_B, 1, 1))
    sub = lax.broadcasted_iota(jnp.int32, (_B, 8, 1), 1)
    g = jnp.sum(jnp.where(sub == r, blocks, 0.0), axis=1)
    tgt = jnp.sum(xn * g, axis=1, keepdims=True) * _INV_TEMP
    nll = _SHIFT + jnp.log(s_ref[...]) - tgt
    out_ref[0, 0] = jnp.sum(nll) * (1.0 / _B)


def _tc_combine(inputs, gathered_blocks, targets, s, interpret=False):
    return pl.pallas_call(
        _tc_combine_kernel,
        out_specs=pl.BlockSpec(memory_space=pltpu.SMEM),
        out_shape=jax.ShapeDtypeStruct((1, 1), jnp.float32),
        interpret=interpret,
    )(inputs, gathered_blocks, targets, s)


def kernel(inputs, targets, features):
    idx = targets.astype(jnp.int32)
    features_bf = _tc_convert(features)
    gathered_blocks = _sc_gather_blocks(features_bf, idx)
    s = _tc_main(inputs, features_bf)
    out = _tc_combine(inputs, gathered_blocks, idx, s)
    return out[0, 0]


# speedup vs baseline: 1.3173x; 1.3173x over previous
import functools

import jax
import jax.numpy as jnp
from jax import lax
from jax.experimental import pallas as pl
from jax.experimental.pallas import tpu as pltpu
from jax.experimental.pallas import tpu_sc as plsc

_B = 1024
_D = 64
_N = 100000
_INV_TEMP = 20.0
_SHIFT = 20.0
_TILE = 4000
_STEPS = _N // _TILE
_LOG2E = 1.4426950408889634
_C = _SHIFT * _LOG2E

_NC = 2
_NS = 16
_NW = _NC * _NS
_BPW = _B // _NW


def _sc_gather_blocks(table, idx):
    mesh = plsc.ScalarSubcoreMesh(axis_name="c", num_cores=_NC)
    bpc = _B // _NC

    @functools.partial(
        pl.kernel,
        mesh=mesh,
        out_type=jax.ShapeDtypeStruct((8 * _B, _D), jnp.bfloat16),
        scratch_types=[
            pltpu.SMEM((bpc,), jnp.int32),
            pltpu.SemaphoreType.DMA,
        ],
    )
    def gather_k(table_hbm, idx_hbm, out_hbm, idx_s, sem):
        base = lax.axis_index("c") * bpc
        pltpu.async_copy(idx_hbm.at[pl.ds(base, bpc)], idx_s, sem).wait()

        @pl.loop(0, bpc)
        def _(i):
            tb = pl.multiple_of((idx_s[i] >> 3) << 3, 8)
            pltpu.async_copy(table_hbm.at[pl.ds(tb, 8)],
                             out_hbm.at[pl.ds((base + i) * 8, 8)], sem)

        @pl.loop(0, bpc)
        def _(i):
            pltpu.make_async_copy(table_hbm.at[pl.ds(0, 8)],
                                  out_hbm.at[pl.ds(0, 8)], sem).wait()

    return gather_k(table, idx)


def _tc_main_kernel(x_ref, f_ref, s_ref, xs_ref):
    k = pl.program_id(0)

    @pl.when(k == 0)
    def _():
        x = x_ref[...]
        nrm = jnp.sqrt(jnp.sum(x * x, axis=1, keepdims=True))
        xn = x / jnp.maximum(nrm, 1e-12)
        xs_ref[...] = (xn * (_INV_TEMP * _LOG2E)).astype(jnp.bfloat16)
        s_ref[...] = jnp.zeros_like(s_ref)

    logits2 = lax.dot_general(
        xs_ref[...], f_ref[...],
        (((1,), (1,)), ((), ())),
        preferred_element_type=jnp.float32)
    s_ref[...] += jnp.sum(jnp.exp2(logits2 - _C), axis=1, keepdims=True)


def _tc_main(inputs, features, interpret=False):
    return pl.pallas_call(
        _tc_main_kernel,
        grid=(_STEPS,),
        in_specs=[
            pl.BlockSpec((_B, _D), lambda k: (0, 0)),
            pl.BlockSpec((_TILE, _D), lambda k: (k, 0)),
        ],
        out_specs=pl.BlockSpec((_B, 1), lambda k: (0, 0)),
        out_shape=jax.ShapeDtypeStruct((_B, 1), jnp.float32),
        scratch_shapes=[
            pltpu.VMEM((_B, _D), jnp.bfloat16),
        ],
        compiler_params=pltpu.CompilerParams(
            dimension_semantics=("arbitrary",)),
        interpret=interpret,
    )(inputs, features)


def _tc_combine_kernel(x_ref, g_ref, t_ref, s_ref, out_ref):
    x = x_ref[...]
    nrm = jnp.sqrt(jnp.sum(x * x, axis=1, keepdims=True))
    xn = x / jnp.maximum(nrm, 1e-12)
    blocks = jnp.reshape(g_ref[...].astype(jnp.float32), (_B, 8, _D))
    r = jnp.reshape(t_ref[...] & 7, (_B, 1, 1))
    sub = lax.broadcasted_iota(jnp.int32, (_B, 8, 1), 1)
    g = jnp.sum(jnp.where(sub == r, blocks, 0.0), axis=1)
    tgt = jnp.sum(xn * g, axis=1, keepdims=True) * _INV_TEMP
    nll = _SHIFT + jnp.log(s_ref[...]) - tgt
    out_ref[0, 0] = jnp.sum(nll) * (1.0 / _B)


def _tc_combine(inputs, gathered_blocks, targets, s, interpret=False):
    return pl.pallas_call(
        _tc_combine_kernel,
        out_specs=pl.BlockSpec(memory_space=pltpu.SMEM),
        out_shape=jax.ShapeDtypeStruct((1, 1), jnp.float32),
        interpret=interpret,
    )(inputs, gathered_blocks, targets, s)


def kernel(inputs, targets, features):
    idx = targets.astype(jnp.int32)
    features_bf = features.astype(jnp.bfloat16)
    gathered_blocks = _sc_gather_blocks(features_bf, idx)
    s = _tc_main(inputs, features_bf)
    out = _tc_combine(inputs, gathered_blocks, idx, s)
    return out[0, 0]
